# baseline (speedup 1.0000x reference)
import functools

import jax
import jax.numpy as jnp
from jax import lax
from jax.experimental import pallas as pl
from jax.experimental.pallas import tpu as pltpu
from jax.experimental.pallas import tpu_sc as plsc

VOCAB_CAT = 48
VOCAB_ATTR = 102
N_WORDS = 20

NC = 2
NS = 16
L = 16
NW = NC * NS


def _make_sc_call(M):
    assert M % NW == 0
    rpw = M // NW
    blk = 64
    assert rpw % blk == 0
    nblk = rpw // blk
    mesh = plsc.VectorSubcoreMesh(core_axis_name="c", subcore_axis_name="s")

    @functools.partial(
        pl.kernel,
        mesh=mesh,
        compiler_params=pltpu.CompilerParams(needs_layout_passes=False),
        out_type=[
            jax.ShapeDtypeStruct((M * VOCAB_CAT,), jnp.float32),
            jax.ShapeDtypeStruct((M * VOCAB_ATTR,), jnp.float32),
        ],
        scratch_types=[
            pltpu.VMEM((rpw,), jnp.int32),
            pltpu.VMEM((rpw * N_WORDS,), jnp.int32),
            pltpu.VMEM((rpw * VOCAB_CAT,), jnp.float32),
            pltpu.VMEM((rpw * VOCAB_ATTR,), jnp.float32),
            pltpu.SemaphoreType.DMA,
            pltpu.SemaphoreType.DMA,
        ],
    )
    def sc_call(cat_hbm, attr_hbm, out1_hbm, out2_hbm,
                cat_v, attr_v, o1_v, o2_v, sem1, sem2):
        c = lax.axis_index("c")
        s = lax.axis_index("s")
        wid = s * NC + c
        base = wid * rpw

        pltpu.sync_copy(cat_hbm.at[pl.ds(base, rpw)], cat_v)
        pltpu.sync_copy(attr_hbm.at[pl.ds(base * N_WORDS, rpw * N_WORDS)],
                        attr_v)

        iota = lax.iota(jnp.int32, L)
        ones = jnp.full((L,), 1.0, jnp.float32)
        zeros = jnp.zeros((L,), jnp.float32)

        def block(g, carry):
            r0 = g * blk
            o1_off = r0 * VOCAB_CAT
            o2_off = r0 * VOCAB_ATTR
            for j in range(blk * VOCAB_CAT // L):
                o1_v[pl.ds(o1_off + j * L, L)] = zeros
            for j in range(blk * VOCAB_ATTR // L):
                o2_v[pl.ds(o2_off + j * L, L)] = zeros
            for sub in range(blk // L):
                r = r0 + sub * L
                ri = iota + r
                catv = cat_v[pl.ds(r, L)]
                plsc.store_scatter(o1_v, [ri * VOCAB_CAT + catv], ones)
                ri_a = ri * N_WORDS
                ri_o = ri * VOCAB_ATTR
                for w in range(N_WORDS):
                    av = plsc.load_gather(attr_v, [ri_a + w])
                    plsc.store_scatter(o2_v, [ri_o + av], ones)
            pltpu.async_copy(
                o1_v.at[pl.ds(o1_off, blk * VOCAB_CAT)],
                out1_hbm.at[pl.ds(base * VOCAB_CAT + o1_off,
                                  blk * VOCAB_CAT)],
                sem1)
            pltpu.async_copy(
                o2_v.at[pl.ds(o2_off, blk * VOCAB_ATTR)],
                out2_hbm.at[pl.ds(base * VOCAB_ATTR + o2_off,
                                  blk * VOCAB_ATTR)],
                sem2)
            return carry

        lax.fori_loop(0, nblk, block, 0)

        for _ in range(nblk):
            pltpu.make_async_copy(
                o1_v.at[pl.ds(0, blk * VOCAB_CAT)],
                out1_hbm.at[pl.ds(base * VOCAB_CAT, blk * VOCAB_CAT)],
                sem1).wait()
            pltpu.make_async_copy(
                o2_v.at[pl.ds(0, blk * VOCAB_ATTR)],
                out2_hbm.at[pl.ds(base * VOCAB_ATTR, blk * VOCAB_ATTR)],
                sem2).wait()

    return sc_call


@jax.jit
def kernel(category, attributes):
    B, N, _ = category.shape
    M = B * N
    cat_flat = category.reshape(M)
    attr_flat = attributes.reshape(M * N_WORDS)
    o1, o2 = _make_sc_call(M)(cat_flat, attr_flat)
    return (o1.reshape(B, N, VOCAB_CAT), o2.reshape(B, N, VOCAB_ATTR))
